# Initial kernel scaffold; baseline (speedup 1.0000x reference)
import jax
import jax.numpy as jnp
from jax.experimental import pallas as pl


def kernel(x, W1, g1, b1, W2, g2, b2, W3, g3, b3, W4, g4, b4, W5, g5, b5, L1, g6, b6, L2, L2b, g7, b7, L3, L3b):
    raise NotImplementedError("write your pallas kernel here")


# speedup vs baseline: 1.3100x; 1.3100x over previous
import functools

import jax
import jax.numpy as jnp
import numpy as np
from jax import lax
from jax.experimental import pallas as pl
from jax.experimental.pallas import tpu as pltpu
from jax.experimental.pallas import tpu_sc as plsc

_K = 20
_B = 8
_N = 1024
_M = _B * _N
_NEG = -3.0e38


def _lrelu(z):
    return jnp.where(z >= 0, z, 0.2 * z)


def _canon_coords(pc):
    nk = pc.shape[1]
    centered = pc - jnp.mean(pc, axis=1, keepdims=True)
    cov = jnp.einsum('mki,mkj->mij', centered, centered) / (nk - 1)
    _, vecs = jnp.linalg.eigh(cov)
    vecs = vecs[:, :, ::-1]
    max_idx = jnp.argmax(jnp.abs(vecs), axis=1, keepdims=True)
    max_vals = jnp.take_along_axis(vecs, max_idx, axis=1)
    signs = jnp.sign(max_vals)
    signs = jnp.where(signs == 0, jnp.ones_like(signs), signs)
    vecs = vecs * signs
    det = jnp.linalg.det(vecs)
    flip = (det < 0).astype(vecs.dtype)
    col = 1.0 - 2.0 * flip
    scale = jnp.stack([jnp.ones_like(col), jnp.ones_like(col), col], axis=-1)
    vecs = vecs * scale[:, None, :]
    cp = jnp.einsum('mki,mij->mkj', centered, vecs)
    skew = jnp.mean(cp ** 3, axis=1)
    s = jnp.sign(skew)
    s = jnp.where(s == 0, jnp.ones_like(s), s)
    odd = (jnp.sum((s < 0).astype(jnp.int32), axis=-1) % 2) == 1
    fix = jnp.stack([jnp.ones(odd.shape, s.dtype), jnp.ones(odd.shape, s.dtype),
                     jnp.where(odd, -1.0, 1.0).astype(s.dtype)], axis=-1)
    s = s * fix
    return cp * s[:, None, :]


def _cloud_order_body(dr_ref, dc_ref, inv_ref):
    dr = dr_ref[0]
    dc = dc_ref[0]
    n = dr.shape[0]
    r = [dr[:, j:j + 1] for j in range(3)]
    c = [dc[j:j + 1, :] for j in range(3)]
    ii = lax.broadcasted_iota(jnp.int32, (n, n), 0)
    jj = lax.broadcasted_iota(jnp.int32, (n, n), 1)
    prec = ((r[0] < c[0])
            | ((r[0] == c[0])
               & ((r[1] < c[1])
                  | ((r[1] == c[1])
                     & ((r[2] < c[2]) | ((r[2] == c[2]) & (ii < jj)))))))
    rank = jnp.sum(prec.astype(jnp.int32), axis=0, keepdims=True)
    sel = ii == rank
    inv = jnp.sum(jnp.where(sel, jj, 0), axis=1, keepdims=True)
    inv_ref[0] = inv + pl.program_id(0) * n


def _cloud_order_call(cp2):
    cp2T = jnp.transpose(cp2, (0, 2, 1))
    return pl.pallas_call(
        _cloud_order_body,
        grid=(_B,),
        in_specs=[pl.BlockSpec((1, _N, 3), lambda b: (b, 0, 0)),
                  pl.BlockSpec((1, 3, _N), lambda b: (b, 0, 0))],
        out_specs=pl.BlockSpec((1, _N, 1), lambda b: (b, 0, 0)),
        out_shape=jax.ShapeDtypeStruct((_B, _N, 1), jnp.int32),
    )(cp2, cp2T)


def _topk_extract(pd, n, bias):
    jj = lax.broadcasted_iota(jnp.int32, (n, n), 1)
    cols = []
    for _ in range(_K):
        m = jnp.max(pd, axis=1, keepdims=True)
        hit = pd == m
        idxj = jnp.min(jnp.where(hit, jj, n), axis=1, keepdims=True)
        cols.append(idxj + bias)
        pd = jnp.where(jj == idxj, _NEG, pd)
    return jnp.concatenate(cols, axis=1)


def _topk_body(pd_ref, out_ref):
    pd = pd_ref[0]
    out_ref[0] = _topk_extract(pd, pd.shape[0], pl.program_id(0) * pd.shape[0])


def _topk_call(pd):
    return pl.pallas_call(
        _topk_body,
        grid=(_B,),
        in_specs=[pl.BlockSpec((1, _N, _N), lambda b: (b, 0, 0))],
        out_specs=pl.BlockSpec((1, _N, _K), lambda b: (b, 0, 0)),
        out_shape=jax.ShapeDtypeStruct((_B, _N, _K), jnp.int32),
    )(pd)


def _knn_body(x_ref, xt_ref, out_ref):
    xb = x_ref[0]
    xc = xt_ref[0]
    n = xb.shape[0]
    g = lax.dot_general(xb, xb, (((1,), (1,)), ((), ())),
                        preferred_element_type=jnp.float32)
    inner = -2.0 * g
    xx_c = jnp.sum(xc * xc, axis=0, keepdims=True)
    xx_r = jnp.transpose(xx_c, (1, 0))
    pd = (-xx_c) - inner - xx_r
    out_ref[0] = _topk_extract(pd, n, pl.program_id(0) * n)


def _knn_call(x):
    c = x.shape[2]
    xt = jnp.transpose(x, (0, 2, 1))
    return pl.pallas_call(
        _knn_body,
        grid=(_B,),
        in_specs=[pl.BlockSpec((1, _N, c), lambda b: (b, 0, 0)),
                  pl.BlockSpec((1, c, _N), lambda b: (b, 0, 0))],
        out_specs=pl.BlockSpec((1, _N, _K), lambda b: (b, 0, 0)),
        out_shape=jax.ShapeDtypeStruct((_B, _N, _K), jnp.int32),
    )(x, xt)


_PB = 512


def _patch_order_body(d_ref, idx_ref, canon_ref, perm_ref):
    d = [d_ref[:, j, :] for j in range(3)]
    nk = d[0].shape[0]
    kk = lax.broadcasted_iota(jnp.int32, d[0].shape, 0)
    ranks = []
    for i in range(nk):
        e = [d[j][i:i + 1] for j in range(3)]
        lt = ((d[0] < e[0])
              | ((d[0] == e[0])
                 & ((d[1] < e[1])
                    | ((d[1] == e[1])
                       & ((d[2] < e[2]) | ((d[2] == e[2]) & (kk < i)))))))
        ranks.append(jnp.sum(lt.astype(jnp.int32), axis=0, keepdims=True))
    rank = jnp.concatenate(ranks, axis=0)
    idx = idx_ref[...]
    oc = [[], [], []]
    op = []
    for r in range(nk):
        msk = rank == r
        mf = msk.astype(jnp.float32)
        for j in range(3):
            oc[j].append(jnp.sum(mf * d[j], axis=0, keepdims=True))
        op.append(jnp.sum(jnp.where(msk, idx, 0), axis=0, keepdims=True))
    ocs = [jnp.concatenate(oc[j], axis=0)[:, None, :] for j in range(3)]
    canon_ref[...] = jnp.concatenate(ocs, axis=1)
    perm_ref[...] = jnp.concatenate(op, axis=0)


def _patch_order_call(cp2T, idxT):
    return pl.pallas_call(
        _patch_order_body,
        grid=(_M // _PB,),
        in_specs=[pl.BlockSpec((_K, 3, _PB), lambda i: (0, 0, i)),
                  pl.BlockSpec((_K, _PB), lambda i: (0, i))],
        out_specs=[pl.BlockSpec((_K, 3, _PB), lambda i: (0, 0, i)),
                   pl.BlockSpec((_K, _PB), lambda i: (0, i))],
        out_shape=[jax.ShapeDtypeStruct((_K, 3, _M), jnp.float32),
                   jax.ShapeDtypeStruct((_K, _M), jnp.int32)],
    )(cp2T, idxT)


def _sc_gather(table, idx):
    rows, dim = idx.shape[0], table.shape[1]
    info = plsc.get_sparse_core_info()
    nw = info.num_cores * info.num_subcores
    per_w = rows // nw
    chunk = min(per_w, 512)
    nch = per_w // chunk
    mesh = plsc.VectorSubcoreMesh(core_axis_name="c", subcore_axis_name="s")

    @functools.partial(
        pl.kernel, mesh=mesh,
        out_type=jax.ShapeDtypeStruct((rows, dim), jnp.float32),
        compiler_params=pltpu.CompilerParams(use_tc_tiling_on_sc=False),
        scratch_types=[
            pltpu.VMEM((chunk,), jnp.int32),
            pltpu.VMEM((chunk, dim), jnp.float32),
            pltpu.SemaphoreType.DMA,
        ],
    )
    def gather_k(table_hbm, idx_hbm, out_hbm, idx_v, rows_v, sem):
        wid = lax.axis_index("s") * info.num_cores + lax.axis_index("c")
        base = wid * per_w
        for t in range(nch):
            off = base + t * chunk
            pltpu.sync_copy(idx_hbm.at[pl.ds(off, chunk)], idx_v)
            pltpu.async_copy(table_hbm.at[idx_v], rows_v, sem).wait()
            pltpu.sync_copy(rows_v, out_hbm.at[pl.ds(off, chunk)])

    return gather_k(table, idx)


_CB = 512


def _conv_body(*refs):
    nparts = (len(refs) - 4) // 2
    xs = refs[:nparts]
    ws = refs[nparts:2 * nparts]
    sc_ref, g_ref, b_ref, out_ref = refs[2 * nparts:]
    acc = None
    for x_ref, w_ref in zip(xs, ws):
        part = lax.dot_general(x_ref[...], w_ref[...], (((1,), (0,)), ((), ())),
                               preferred_element_type=jnp.float32)
        acc = part if acc is None else acc + part
    z = acc * sc_ref[...] * g_ref[...] + b_ref[...]
    out_ref[...] = _lrelu(z)


def _conv_call(parts, wts, sc, g, b):
    out_c = wts[0].shape[1]
    grid = (_M // _CB,)
    in_specs = [pl.BlockSpec((_CB, p.shape[1]), lambda i: (i, 0)) for p in parts]
    in_specs += [pl.BlockSpec(w.shape, lambda i: (0, 0)) for w in wts]
    in_specs += [pl.BlockSpec((1, 1), lambda i: (0, 0))]
    in_specs += [pl.BlockSpec((1, out_c), lambda i: (0, 0))] * 2
    return pl.pallas_call(
        _conv_body,
        grid=grid,
        in_specs=in_specs,
        out_specs=pl.BlockSpec((_CB, out_c), lambda i: (i, 0)),
        out_shape=jax.ShapeDtypeStruct((_M, out_c), jnp.float32),
    )(*parts, *wts, sc, g.reshape(1, -1), b.reshape(1, -1))


def _head_body(x1, x2, x3, x4, wa, wb, wc, wd, sc_ref, g5, b5, l1t, g6, b6,
               l2t, l2b, g7, b7, l3t, l3b, out_ref):
    dn = (((1,), (0,)), ((), ()))
    sc = sc_ref[...]
    z = (lax.dot_general(x1[0], wa[...], dn, preferred_element_type=jnp.float32)
         + lax.dot_general(x2[0], wb[...], dn, preferred_element_type=jnp.float32)
         + lax.dot_general(x3[0], wc[...], dn, preferred_element_type=jnp.float32)
         + lax.dot_general(x4[0], wd[...], dn, preferred_element_type=jnp.float32))
    xo = _lrelu(z * sc * g5[...] + b5[...])
    xmax = jnp.max(xo, axis=0, keepdims=True)
    xmean = jnp.sum(xo, axis=0, keepdims=True) / xo.shape[0]
    xp = jnp.concatenate([xmax, xmean], axis=1)
    h = _lrelu(lax.dot_general(xp, l1t[...], dn,
                               preferred_element_type=jnp.float32)
               * sc * g6[...] + b6[...])
    h2 = _lrelu((lax.dot_general(h, l2t[...], dn,
                                 preferred_element_type=jnp.float32)
                 + l2b[...]) * sc * g7[...] + b7[...])
    out_ref[0] = (lax.dot_general(h2, l3t[...], dn,
                                  preferred_element_type=jnp.float32)
                  + l3b[...])


def _head_call(xs, w5s, sc, g5, b5, l1t, g6, b6, l2t, l2b, g7, b7, l3t, l3b):
    args = []
    in_specs = []
    for xf in xs:
        c = xf.shape[2]
        args.append(xf)
        in_specs.append(pl.BlockSpec((1, _N, c), lambda bb: (bb, 0, 0)))
    smalls = [*w5s, sc, g5.reshape(1, -1), b5.reshape(1, -1), l1t,
              g6.reshape(1, -1), b6.reshape(1, -1), l2t, l2b.reshape(1, -1),
              g7.reshape(1, -1), b7.reshape(1, -1), l3t, l3b.reshape(1, -1)]
    for a in smalls:
        args.append(a)
        in_specs.append(pl.BlockSpec(a.shape, lambda bb: (0, 0)))
    return pl.pallas_call(
        _head_body,
        grid=(_B,),
        in_specs=in_specs,
        out_specs=pl.BlockSpec((1, 1, 40), lambda bb: (bb, 0, 0)),
        out_shape=jax.ShapeDtypeStruct((_B, 1, 40), jnp.float32),
    )(*args).reshape(_B, 40)


def _split_w(w, c):
    f = 3 + c
    cidx = np.array([t * f + j for t in range(_K) for j in range(3)])
    aidx = np.array([t * f + 3 + j for t in range(_K) for j in range(c)])
    return jnp.transpose(w[:, cidx]), jnp.transpose(w[:, aidx])


def _pad16(a):
    return jnp.concatenate([a, jnp.zeros((a.shape[0], 13), jnp.float32)], axis=1)


def _patch_stage(pts_pad, idxg):
    idx_flat = idxg.reshape(_M * _K)
    pp = _sc_gather(pts_pad, idx_flat)
    patch_pts = pp.reshape(_M, _K, 16)[:, :, :3]
    cp2 = _canon_coords(patch_pts)
    cp2T = jnp.transpose(cp2, (1, 2, 0))
    idxT = jnp.transpose(idxg.reshape(_M, _K), (1, 0))
    canonT, permT = _patch_order_call(cp2T, idxT)
    canon3 = jnp.transpose(canonT, (2, 0, 1))
    perm_flat = jnp.transpose(permT, (1, 0)).reshape(_M * _K)
    return canon3, perm_flat


def kernel(x, W1, g1, b1, W2, g2, b2, W3, g3, b3, W4, g4, b4, W5, g5, b5,
           L1, g6, b6, L2, L2b, g7, b7, L3, L3b):
    sc = (1.0 / jnp.sqrt(1.0 + jnp.float32(1e-5))).reshape(1, 1)

    pc = jnp.transpose(x, (0, 2, 1))
    cp2_cloud = _canon_coords(pc)
    inv = _cloud_order_call(cp2_cloud).reshape(_M)
    cloud_tab = _pad16(cp2_cloud.reshape(_M, 3))
    pts_pad = _sc_gather(cloud_tab, inv)

    pts = pts_pad[:, :3].reshape(_B, _N, 3)
    x0 = jnp.transpose(pts, (0, 2, 1))
    inner = -2.0 * jnp.einsum('bcn,bcm->bnm', x0, x0)
    xx = jnp.sum(x0 ** 2, axis=1, keepdims=True)
    pd1 = -xx - inner - jnp.transpose(xx, (0, 2, 1))
    idxg = _topk_call(pd1)
    canon3, _ = _patch_stage(pts_pad, idxg)
    x1 = _conv_call([canon3.reshape(_M, _K * 3)], [jnp.transpose(W1)],
                    sc, g1, b1)

    def stage(x_feat, w, g, b):
        c = x_feat.shape[1]
        idxg_s = _knn_call(x_feat.reshape(_B, _N, c))
        canon3_s, perm_flat = _patch_stage(pts_pad, idxg_s)
        al3 = _sc_gather(x_feat, perm_flat).reshape(_M, _K, c)
        feat = jnp.concatenate([canon3_s, al3], axis=2).reshape(_M, _K * (3 + c))
        return _conv_call([feat], [jnp.transpose(w)], sc, g, b)

    x2 = stage(x1, W2, g2, b2)
    x3 = stage(x2, W3, g3, b3)
    x4 = stage(x3, W4, g4, b4)

    w5s = [jnp.transpose(W5[:, 0:64]), jnp.transpose(W5[:, 64:128]),
           jnp.transpose(W5[:, 128:256]), jnp.transpose(W5[:, 256:512])]
    xs = [x1.reshape(_B, _N, 64), x2.reshape(_B, _N, 64),
          x3.reshape(_B, _N, 128), x4.reshape(_B, _N, 256)]
    return _head_call(xs, w5s, sc, g5, b5, jnp.transpose(L1), g6, b6,
                      jnp.transpose(L2), L2b, g7, b7, jnp.transpose(L3), L3b)
